# baseline (speedup 1.0000x reference)
import functools

import jax
import jax.numpy as jnp
from jax import lax
from jax.experimental import pallas as pl
from jax.experimental.pallas import tpu as pltpu
from jax.experimental.pallas import tpu_sc as plsc

L = 16
NV = 10
L2 = 640
NTILES = 16
KPAD = NTILES * L2


def _make_sc_kernel(tp1, k):
    nbits = (tp1 - 1).bit_length()
    mesh = plsc.VectorSubcoreMesh(core_axis_name="c", subcore_axis_name="s")

    @functools.partial(
        pl.kernel,
        out_type=(
            jax.ShapeDtypeStruct((3, k), jnp.float32),
            jax.ShapeDtypeStruct((4, k), jnp.float32),
        ),
        mesh=mesh,
        compiler_params=pltpu.CompilerParams(needs_layout_passes=False,
                                             use_tc_tiling_on_sc=True),
        scratch_types=[
            pltpu.VMEM((L,), jnp.int32),
            pltpu.VMEM((k,), jnp.float32),
            pltpu.VMEM((k,), jnp.float32),
            pltpu.VMEM((k,), jnp.float32),
            pltpu.VMEM((4 * NV * L2,), jnp.float32),
            pltpu.VMEM((4 * L2,), jnp.float32),
            pltpu.VMEM_SHARED((4 * NV * KPAD,), jnp.float32),
            pltpu.VMEM_SHARED((4 * KPAD,), jnp.float32),
            pltpu.SemaphoreType.DMA,
            pltpu.SemaphoreType.DMA,
            pltpu.SemaphoreType.DMA,
            pltpu.SemaphoreType.DMA,
        ],
    )
    def fenwick_sc(tvec_hbm, trans_hbm, rots_hbm, out_t_hbm, out_r_hbm,
                   tv_v, rbuf0, rbuf1, rbuf2, loc_rows, loc_out,
                   sp_rows, sp_out, sem0, sem1, sem2, semp):
        cid = lax.axis_index("c")
        sid = lax.axis_index("s")
        rbufs = [rbuf0, rbuf1, rbuf2]
        sems = [sem0, sem1, sem2]

        pltpu.sync_copy(tvec_hbm, tv_v)
        iot = lax.iota(jnp.int32, L)
        tt = tv_v[...]
        idxs = jnp.zeros((L,), jnp.int32)
        for i in range(nbits):
            idxs = jnp.where(iot == i, tt, idxs)
            tt = tt - (tt & (-tt))
        r_list = [jnp.sum(jnp.where(iot == i, idxs, 0)) for i in range(NV)]
        w_list = [jnp.where(r > 0, 1.0, 0.0).astype(jnp.float32)
                  for r in r_list]

        def run_side(nplanes, row_src, out_hbm):
            units = [(p, i) for p in range(nplanes) for i in range(NV)]

            for v, (p, i) in enumerate(units):
                @pl.when(v % NTILES == sid)
                def _(p=p, i=i, b=v // NTILES):
                    pltpu.async_copy(row_src(p, r_list[i]), rbufs[b],
                                     sems[b])
            for v, (p, i) in enumerate(units):
                @pl.when(v % NTILES == sid)
                def _(p=p, i=i, b=v // NTILES):
                    pltpu.make_async_copy(row_src(p, r_list[i]), rbufs[b],
                                          sems[b]).wait()
                    pltpu.async_copy(
                        rbufs[b],
                        sp_rows.at[pl.ds((p * NV + i) * KPAD, k)], semp)
            for v, (p, i) in enumerate(units):
                @pl.when(v % NTILES == sid)
                def _(p=p, i=i, b=v // NTILES):
                    pltpu.make_async_copy(
                        rbufs[b],
                        sp_rows.at[pl.ds((p * NV + i) * KPAD, k)],
                        semp).wait()
            plsc.subcore_barrier()

            kc = sid * L2
            for p, i in units:
                pltpu.async_copy(
                    sp_rows.at[pl.ds((p * NV + i) * KPAD + kc, L2)],
                    loc_rows.at[pl.ds((p * NV + i) * L2, L2)], sem0)
            for p, i in units:
                pltpu.make_async_copy(
                    sp_rows.at[pl.ds((p * NV + i) * KPAD + kc, L2)],
                    loc_rows.at[pl.ds((p * NV + i) * L2, L2)], sem0).wait()

            @plsc.parallel_loop(0, L2, step=L, unroll=4)
            def chunk_body(off):
                s = []
                for p in range(nplanes):
                    acc = (loc_rows[pl.ds(p * NV * L2 + off, L)]
                           * w_list[0])
                    for i in range(1, NV):
                        acc = acc + (loc_rows[pl.ds((p * NV + i) * L2 + off,
                                                    L)] * w_list[i])
                    s.append(acc)
                if nplanes == 4:
                    n2 = s[0] * s[0] + s[1] * s[1] + s[2] * s[2] + s[3] * s[3]
                    n2 = jnp.maximum(n2, jnp.float32(1e-24))
                    est = plsc.bitcast(
                        jnp.int32(0x5F3759DF)
                        - (plsc.bitcast(n2, jnp.int32) >> 1), jnp.float32)
                    for _ in range(3):
                        est = est * (1.5 - 0.5 * n2 * est * est)
                    s = [sp * est for sp in s]
                for p in range(nplanes):
                    loc_out[pl.ds(p * L2 + off, L)] = s[p]

            for p in range(nplanes):
                pltpu.async_copy(loc_out.at[pl.ds(p * L2, L2)],
                                 sp_out.at[pl.ds(p * KPAD + kc, L2)], sem1)
            for p in range(nplanes):
                pltpu.make_async_copy(loc_out.at[pl.ds(p * L2, L2)],
                                      sp_out.at[pl.ds(p * KPAD + kc, L2)],
                                      sem1).wait()
            plsc.subcore_barrier()

            @pl.when(sid < nplanes)
            def _():
                pltpu.sync_copy(sp_out.at[pl.ds(sid * KPAD, k)], rbufs[0])
                pltpu.sync_copy(rbufs[0], out_hbm.at[sid])

        @pl.when(cid == 0)
        def _():
            run_side(3, lambda p, r: trans_hbm.at[p, r], out_t_hbm)

        @pl.when(cid == 1)
        def _():
            run_side(4, lambda p, r: rots_hbm.at[r, p], out_r_hbm)

    return fenwick_sc


def kernel(node_translations, node_rotations, t):
    tp1, k = node_translations.shape[0], node_translations.shape[1]
    trans_T = jnp.transpose(node_translations, (2, 0, 1))
    rots_T = jnp.transpose(node_rotations, (0, 2, 1))
    tvec = jnp.broadcast_to(jnp.asarray(t, jnp.int32), (L,))
    out_t3, out_r4 = _make_sc_kernel(tp1, k)(tvec, trans_T, rots_T)
    return jnp.transpose(out_t3, (1, 0)), jnp.transpose(out_r4, (1, 0))


# speedup vs baseline: 1.0147x; 1.0147x over previous
import functools

import jax
import jax.numpy as jnp
from jax import lax
from jax.experimental import pallas as pl
from jax.experimental.pallas import tpu as pltpu
from jax.experimental.pallas import tpu_sc as plsc

L = 16
NV = 10
L2 = 640
NTILES = 16
KPAD = NTILES * L2


def _make_sc_kernel(tp1, k):
    nbits = (tp1 - 1).bit_length()
    mesh = plsc.VectorSubcoreMesh(core_axis_name="c", subcore_axis_name="s")

    @functools.partial(
        pl.kernel,
        out_type=(
            jax.ShapeDtypeStruct((3, k), jnp.float32),
            jax.ShapeDtypeStruct((4, k), jnp.float32),
        ),
        mesh=mesh,
        compiler_params=pltpu.CompilerParams(needs_layout_passes=False,
                                             use_tc_tiling_on_sc=True),
        scratch_types=[
            pltpu.VMEM((L,), jnp.int32),
            pltpu.VMEM((k,), jnp.float32),
            pltpu.VMEM((k,), jnp.float32),
            pltpu.VMEM((k,), jnp.float32),
            pltpu.VMEM((4 * NV * L2,), jnp.float32),
            pltpu.VMEM((4 * L2,), jnp.float32),
            pltpu.VMEM_SHARED((4 * NV * KPAD,), jnp.float32),
            pltpu.VMEM_SHARED((4 * KPAD,), jnp.float32),
            pltpu.SemaphoreType.DMA,
            pltpu.SemaphoreType.DMA,
            pltpu.SemaphoreType.DMA,
            pltpu.SemaphoreType.DMA,
        ],
    )
    def fenwick_sc(tvec_hbm, trans_hbm, rots_hbm, out_t_hbm, out_r_hbm,
                   tv_v, rbuf0, rbuf1, rbuf2, loc_rows, loc_out,
                   sp_rows, sp_out, sem0, sem1, sem2, semp):
        cid = lax.axis_index("c")
        sid = lax.axis_index("s")
        rbufs = [rbuf0, rbuf1, rbuf2]
        sems = [sem0, sem1, sem2]

        pltpu.sync_copy(tvec_hbm, tv_v)
        iot = lax.iota(jnp.int32, L)
        tt = tv_v[...]
        idxs = jnp.zeros((L,), jnp.int32)
        for i in range(nbits):
            idxs = jnp.where(iot == i, tt, idxs)
            tt = tt - (tt & (-tt))
        r_list = [jnp.sum(jnp.where(iot == i, idxs, 0)) for i in range(NV)]
        w_list = [jnp.where(r > 0, 1.0, 0.0).astype(jnp.float32)
                  for r in r_list]

        def run_side(nplanes, row_src, out_hbm):
            units = [(p, i) for p in range(nplanes) for i in range(NV)]

            for v, (p, i) in enumerate(units):
                @pl.when(v % NTILES == sid)
                def _(p=p, i=i, b=v // NTILES):
                    pltpu.async_copy(row_src(p, r_list[i]), rbufs[b],
                                     sems[b])
            for v, (p, i) in enumerate(units):
                @pl.when(v % NTILES == sid)
                def _(p=p, i=i, b=v // NTILES):
                    pltpu.make_async_copy(row_src(p, r_list[i]), rbufs[b],
                                          sems[b]).wait()
                    pltpu.async_copy(
                        rbufs[b],
                        sp_rows.at[pl.ds((p * NV + i) * KPAD, k)], semp)
            for v, (p, i) in enumerate(units):
                @pl.when(v % NTILES == sid)
                def _(p=p, i=i, b=v // NTILES):
                    pltpu.make_async_copy(
                        rbufs[b],
                        sp_rows.at[pl.ds((p * NV + i) * KPAD, k)],
                        semp).wait()
            plsc.subcore_barrier()

            kc = sid * L2
            for p, i in units:
                pltpu.async_copy(
                    sp_rows.at[pl.ds((p * NV + i) * KPAD + kc, L2)],
                    loc_rows.at[pl.ds((p * NV + i) * L2, L2)], sem0)
            for p, i in units:
                pltpu.make_async_copy(
                    sp_rows.at[pl.ds((p * NV + i) * KPAD + kc, L2)],
                    loc_rows.at[pl.ds((p * NV + i) * L2, L2)], sem0).wait()

            @plsc.parallel_loop(0, L2, step=L, unroll=2)
            def chunk_body(off):
                s = []
                for p in range(nplanes):
                    acc = (loc_rows[pl.ds(p * NV * L2 + off, L)]
                           * w_list[0])
                    for i in range(1, NV):
                        acc = acc + (loc_rows[pl.ds((p * NV + i) * L2 + off,
                                                    L)] * w_list[i])
                    s.append(acc)
                if nplanes == 4:
                    n2 = s[0] * s[0] + s[1] * s[1] + s[2] * s[2] + s[3] * s[3]
                    n2 = jnp.maximum(n2, jnp.float32(1e-24))
                    est = plsc.bitcast(
                        jnp.int32(0x5F3759DF)
                        - (plsc.bitcast(n2, jnp.int32) >> 1), jnp.float32)
                    for _ in range(3):
                        est = est * (1.5 - 0.5 * n2 * est * est)
                    s = [sp * est for sp in s]
                for p in range(nplanes):
                    loc_out[pl.ds(p * L2 + off, L)] = s[p]

            for p in range(nplanes):
                pltpu.async_copy(loc_out.at[pl.ds(p * L2, L2)],
                                 sp_out.at[pl.ds(p * KPAD + kc, L2)], sem1)
            for p in range(nplanes):
                pltpu.make_async_copy(loc_out.at[pl.ds(p * L2, L2)],
                                      sp_out.at[pl.ds(p * KPAD + kc, L2)],
                                      sem1).wait()
            plsc.subcore_barrier()

            @pl.when(sid < nplanes)
            def _():
                pltpu.sync_copy(sp_out.at[pl.ds(sid * KPAD, k)], rbufs[0])
                pltpu.sync_copy(rbufs[0], out_hbm.at[sid])

        @pl.when(cid == 0)
        def _():
            run_side(3, lambda p, r: trans_hbm.at[p, r], out_t_hbm)

        @pl.when(cid == 1)
        def _():
            run_side(4, lambda p, r: rots_hbm.at[r, p], out_r_hbm)

    return fenwick_sc


def kernel(node_translations, node_rotations, t):
    tp1, k = node_translations.shape[0], node_translations.shape[1]
    trans_T = jnp.transpose(node_translations, (2, 0, 1))
    rots_T = jnp.transpose(node_rotations, (0, 2, 1))
    tvec = jnp.broadcast_to(jnp.asarray(t, jnp.int32), (L,))
    out_t3, out_r4 = _make_sc_kernel(tp1, k)(tvec, trans_T, rots_T)
    return jnp.transpose(out_t3, (1, 0)), jnp.transpose(out_r4, (1, 0))
